# baseline (speedup 1.0000x reference)
import functools

import jax
import jax.numpy as jnp
from jax import lax
from jax.experimental import pallas as pl
from jax.experimental.pallas import tpu as pltpu
from jax.experimental.pallas import tpu_sc as plsc

N = 10000
D_IN = 128
D_HID = 128
D_OUT = 40
D_OUTP = 48

NC, NS = 2, 16
NW = NC * NS
CHUNK = 128
E = 320000
CPW = -(-E // (NW * CHUNK))
CPW += CPW % 2
E_PAD = NW * CPW * CHUNK
NCHUNKS = E_PAD // CHUNK
PAD_DST = N

HALFN = 5008
N_SH = 5024
JUNK0 = 5008
CPW2 = NCHUNKS // NS

_mesh = plsc.VectorSubcoreMesh(core_axis_name="c", subcore_axis_name="s")


def _fill_rows(buf, rows, d, value):
    def body(r, _):
        for k in range(d // 16):
            buf[r, pl.ds(k * 16, 16)] = jnp.full((16,), value, jnp.float32)
        return 0
    lax.fori_loop(0, rows, body, 0)


def _mask_to_junk(didx, lo, span):
    junk = JUNK0 + lax.iota(jnp.int32, 16)
    def body(j, _):
        for k in range(CHUNK // 16):
            v = didx[j, pl.ds(k * 16, 16)] - lo
            ok = (v >= 0) & (v < span)
            didx[j, pl.ds(k * 16, 16)] = jnp.where(ok, v, junk)
        return 0
    lax.fori_loop(0, CPW2, body, 0)


def _make_deg():
    @functools.partial(
        pl.kernel,
        mesh=_mesh,
        out_type=jax.ShapeDtypeStruct((N, 16), jnp.float32),
        scratch_types=[
            pltpu.VMEM((CPW2, CHUNK), jnp.int32),
            pltpu.VMEM((CHUNK, 16), jnp.float32),
            pltpu.VMEM((CHUNK, 16), jnp.float32),
            pltpu.VMEM_SHARED((N_SH, 16), jnp.float32),
        ],
    )
    def deg_kernel(dst_hbm, out_hbm, didx, ones_v, zeros_v, deg_sh):
        cid = lax.axis_index("c")
        sid = lax.axis_index("s")

        _fill_rows(ones_v, CHUNK, 16, 1.0)
        _fill_rows(zeros_v, CHUNK, 16, 0.0)

        z0 = jnp.minimum(sid * 320, N_SH - 320)
        for b in range(5):
            pltpu.sync_copy(zeros_v.at[pl.ds(0, 64)],
                            deg_sh.at[pl.ds(z0 + b * 64, 64)])

        pltpu.sync_copy(dst_hbm.at[pl.ds(sid * CPW2, CPW2)], didx)
        lo = cid * HALFN
        span = jnp.where(cid == 0, HALFN, N - HALFN)
        _mask_to_junk(didx, lo, span)
        plsc.subcore_barrier()

        def body(j, _):
            pltpu.sync_copy(ones_v, deg_sh.at[didx.at[j]], add=True)
            return 0
        lax.fori_loop(0, CPW2, body, 0)

        plsc.subcore_barrier()
        e0 = jnp.minimum(sid * 320, span - 320)
        pltpu.sync_copy(deg_sh.at[pl.ds(e0, 320)],
                        out_hbm.at[pl.ds(lo + e0, 320)])

    return deg_kernel


EPT2 = CPW2 * CHUNK


def _make_agg(d, nb):
    @functools.partial(
        pl.kernel,
        mesh=_mesh,
        compiler_params=pltpu.CompilerParams(use_tc_tiling_on_sc=False,
                                             needs_layout_passes=False),
        out_type=jax.ShapeDtypeStruct((N, d), jnp.float32),
        scratch_types=[
            pltpu.VMEM((EPT2 + 2 * CHUNK,), jnp.int32),
            pltpu.VMEM((EPT2 + 2 * CHUNK,), jnp.int32),
            [pltpu.VMEM((CHUNK, d), jnp.float32)] * nb,
            pltpu.VMEM_SHARED((N_SH, d), jnp.float32),
            [pltpu.SemaphoreType.DMA] * nb,
            [pltpu.SemaphoreType.DMA] * nb,
        ],
    )
    def agg_kernel(g_hbm, src_hbm, dst_hbm, out_hbm,
                   sidx_f, didx_f, rows, agg_sh, sem_g, sem_s):
        cid = lax.axis_index("c")
        sid = lax.axis_index("s")

        _fill_rows(rows[0], CHUNK, d, 0.0)
        z0 = jnp.minimum(sid * 320, N_SH - 320)
        for b in range(320 // 64):
            pltpu.sync_copy(rows[0].at[pl.ds(0, 64)],
                            agg_sh.at[pl.ds(z0 + b * 64, 64)])

        t0 = sid * EPT2
        pltpu.sync_copy(src_hbm.at[pl.ds(t0, EPT2)], sidx_f.at[pl.ds(0, EPT2)])
        pltpu.sync_copy(dst_hbm.at[pl.ds(t0, EPT2)], didx_f.at[pl.ds(0, EPT2)])
        lo = cid * HALFN
        span = jnp.where(cid == 0, HALFN, N - HALFN)

        junk = JUNK0 + lax.iota(jnp.int32, 16)
        trash = EPT2 + CHUNK + lax.iota(jnp.int32, 16)
        def comp(g, cnt):
            v = didx_f[pl.ds(g * 16, 16)]
            s = sidx_f[pl.ds(g * 16, 16)]
            rel = v - lo
            ok = (rel >= 0) & (rel < span)
            oki = ok.astype(jnp.int32)
            c = plsc.cumsum(oki)
            pos = jnp.where(ok, cnt + c - 1, trash)
            plsc.store_scatter(didx_f, [pos], rel)
            plsc.store_scatter(sidx_f, [pos], s)
            return cnt + c[15]
        cnt = lax.fori_loop(0, EPT2 // 16, comp, jnp.int32(0))

        for t in range(CHUNK // 16):
            didx_f[pl.ds(cnt + 16 * t, 16)] = junk
            sidx_f[pl.ds(cnt + 16 * t, 16)] = jnp.zeros((16,), jnp.int32)
        nch = (cnt + CHUNK - 1) >> 7
        if nb == 3:
            nfull = (nch * 43691) >> 17
        else:
            nfull = nch >> 2
        for t in range(CHUNK // 16):
            didx_f[pl.ds(nch * CHUNK + 16 * t, 16)] = junk
        plsc.subcore_barrier()

        jchunk = didx_f.at[pl.ds(nch * CHUNK, CHUNK)]
        for b in range(nb):
            pltpu.async_copy(rows[0], agg_sh.at[jchunk], sem_s[b], add=True)

        def body(i, _):
            j = nb * i
            for b in range(nb):
                pltpu.make_async_copy(rows[b], agg_sh.at[jchunk],
                                      sem_s[b]).wait()
            gs = [pltpu.async_copy(
                      g_hbm.at[sidx_f.at[pl.ds((j + b) * CHUNK, CHUNK)]],
                      rows[b], sem_g[b])
                  for b in range(nb)]
            for b in range(nb):
                gs[b].wait()
                pltpu.async_copy(
                    rows[b],
                    agg_sh.at[didx_f.at[pl.ds((j + b) * CHUNK, CHUNK)]],
                    sem_s[b], add=True)
            return 0
        lax.fori_loop(0, nfull, body, 0)

        def tail(j, _):
            pltpu.make_async_copy(rows[0], agg_sh.at[jchunk], sem_s[0]).wait()
            pltpu.async_copy(g_hbm.at[sidx_f.at[pl.ds(j * CHUNK, CHUNK)]],
                             rows[0], sem_g[0]).wait()
            pltpu.async_copy(rows[0],
                             agg_sh.at[didx_f.at[pl.ds(j * CHUNK, CHUNK)]],
                             sem_s[0], add=True)
            return 0
        lax.fori_loop(nb * nfull, nch, tail, 0)

        for b in range(nb):
            pltpu.make_async_copy(rows[b], agg_sh.at[jchunk], sem_s[b]).wait()

        plsc.subcore_barrier()
        e0 = jnp.minimum(sid * 320, span - 320)
        pltpu.sync_copy(agg_sh.at[pl.ds(e0, 320)],
                        out_hbm.at[pl.ds(lo + e0, 320)])

    return agg_kernel


_deg_call = _make_deg()
_agg_wide = _make_agg(D_HID, 3)
_agg_narrow = _make_agg(D_OUTP, 4)

RB = 2000


def _dis_of(deg_blk):
    return lax.rsqrt(deg_blk[:, 0:1] + 1.0)


def _mm_body(x_ref, w_ref, h_ref):
    h_ref[...] = jnp.dot(x_ref[...], w_ref[...],
                         preferred_element_type=jnp.float32)


def _scale_body(h_ref, deg_ref, g_ref):
    g_ref[...] = h_ref[...] * _dis_of(deg_ref[...])


def _mid_body(agg_ref, g1_ref, deg_ref, b1_ref, w2_ref, g2_ref):
    dis = _dis_of(deg_ref[...])
    h = jnp.maximum(b1_ref[...] + dis * (agg_ref[...] + g1_ref[...]), 0.0)
    g2_ref[...] = jnp.dot(h, w2_ref[...],
                          preferred_element_type=jnp.float32) * dis


def _fin_body(agg_ref, g2_ref, deg_ref, b2_ref, o_ref):
    dis = _dis_of(deg_ref[...])
    o_ref[...] = b2_ref[...] + dis * (agg_ref[...] + g2_ref[...])


def kernel(x, edge_index, W1, b1, W2, b2):
    src = edge_index[0].astype(jnp.int32)
    dst = edge_index[1].astype(jnp.int32)
    src = jnp.concatenate([src, jnp.zeros((E_PAD - E,), jnp.int32)])
    dst = jnp.concatenate([dst, jnp.full((E_PAD - E,), PAD_DST, jnp.int32)])
    src2 = src.reshape(NCHUNKS, CHUNK)
    dst2 = dst.reshape(NCHUNKS, CHUNK)

    dega = _deg_call(dst2)

    h1 = pl.pallas_call(
        _mm_body,
        grid=(N // RB,),
        in_specs=[
            pl.BlockSpec((RB, D_IN), lambda i: (i, 0)),
            pl.BlockSpec((D_IN, D_HID), lambda i: (0, 0)),
        ],
        out_specs=pl.BlockSpec((RB, D_HID), lambda i: (i, 0)),
        out_shape=jax.ShapeDtypeStruct((N, D_HID), jnp.float32),
    )(x, W1)

    deg_spec = pl.BlockSpec((RB, 16), lambda i: (i, 0))
    g1 = pl.pallas_call(
        _scale_body,
        grid=(N // RB,),
        in_specs=[
            pl.BlockSpec((RB, D_HID), lambda i: (i, 0)),
            deg_spec,
        ],
        out_specs=pl.BlockSpec((RB, D_HID), lambda i: (i, 0)),
        out_shape=jax.ShapeDtypeStruct((N, D_HID), jnp.float32),
    )(h1, dega)

    agg1 = _agg_wide(g1, src, dst)

    b1r = b1.reshape(1, D_HID)
    w2p = jnp.pad(W2, ((0, 0), (0, D_OUTP - D_OUT)))
    g2 = pl.pallas_call(
        _mid_body,
        grid=(N // RB,),
        in_specs=[
            pl.BlockSpec((RB, D_HID), lambda i: (i, 0)),
            pl.BlockSpec((RB, D_HID), lambda i: (i, 0)),
            deg_spec,
            pl.BlockSpec((1, D_HID), lambda i: (0, 0)),
            pl.BlockSpec((D_HID, D_OUTP), lambda i: (0, 0)),
        ],
        out_specs=pl.BlockSpec((RB, D_OUTP), lambda i: (i, 0)),
        out_shape=jax.ShapeDtypeStruct((N, D_OUTP), jnp.float32),
    )(agg1, g1, dega, b1r, w2p)

    agg2 = _agg_narrow(g2, src, dst)

    b2p = jnp.pad(b2, (0, D_OUTP - D_OUT)).reshape(1, D_OUTP)
    out = pl.pallas_call(
        _fin_body,
        grid=(N // RB,),
        in_specs=[
            pl.BlockSpec((RB, D_OUTP), lambda i: (i, 0)),
            pl.BlockSpec((RB, D_OUTP), lambda i: (i, 0)),
            deg_spec,
            pl.BlockSpec((1, D_OUTP), lambda i: (0, 0)),
        ],
        out_specs=pl.BlockSpec((RB, D_OUTP), lambda i: (i, 0)),
        out_shape=jax.ShapeDtypeStruct((N, D_OUTP), jnp.float32),
    )(agg2, g2, dega, b2p)

    return out[:, :D_OUT]


# speedup vs baseline: 1.0676x; 1.0676x over previous
import functools

import jax
import jax.numpy as jnp
from jax import lax
from jax.experimental import pallas as pl
from jax.experimental.pallas import tpu as pltpu
from jax.experimental.pallas import tpu_sc as plsc

N = 10000
D_IN = 128
D_HID = 128
D_OUT = 40
D_OUTP = 48

NC, NS = 2, 16
NW = NC * NS
CHUNK = 128
E = 320000
CPW = -(-E // (NW * CHUNK))
CPW += CPW % 2
E_PAD = NW * CPW * CHUNK
NCHUNKS = E_PAD // CHUNK
PAD_DST = N

HALFN = 5008
N_SH = 5024
JUNK0 = 5008
CPW2 = NCHUNKS // NS
EPT2 = CPW2 * CHUNK
LSTW = EPT2 + 2 * CHUNK
CNT_AT = EPT2 + CHUNK

_mesh = plsc.VectorSubcoreMesh(core_axis_name="c", subcore_axis_name="s")


def _fill_rows(buf, rows, d, value):
    def body(r, _):
        for k in range(d // 16):
            buf[r, pl.ds(k * 16, 16)] = jnp.full((16,), value, jnp.float32)
        return 0
    lax.fori_loop(0, rows, body, 0)


def _make_prep():
    @functools.partial(
        pl.kernel,
        mesh=_mesh,
        compiler_params=pltpu.CompilerParams(use_tc_tiling_on_sc=False,
                                             needs_layout_passes=False),
        out_type=(
            jax.ShapeDtypeStruct((N, 16), jnp.float32),
            jax.ShapeDtypeStruct((NC, NS, LSTW), jnp.int32),
            jax.ShapeDtypeStruct((NC, NS, LSTW), jnp.int32),
        ),
        scratch_types=[
            pltpu.VMEM((LSTW,), jnp.int32),
            pltpu.VMEM((LSTW,), jnp.int32),
            pltpu.VMEM((CHUNK, 16), jnp.float32),
            pltpu.VMEM((CHUNK, 16), jnp.float32),
            pltpu.VMEM_SHARED((N_SH, 16), jnp.float32),
        ],
    )
    def prep_kernel(src_hbm, dst_hbm, deg_hbm, csrc_hbm, cdst_hbm,
                    sidx_f, didx_f, ones_v, zeros_v, deg_sh):
        cid = lax.axis_index("c")
        sid = lax.axis_index("s")

        _fill_rows(ones_v, CHUNK, 16, 1.0)
        _fill_rows(zeros_v, CHUNK, 16, 0.0)

        z0 = jnp.minimum(sid * 320, N_SH - 320)
        for b in range(5):
            pltpu.sync_copy(zeros_v.at[pl.ds(0, 64)],
                            deg_sh.at[pl.ds(z0 + b * 64, 64)])

        t0 = sid * EPT2
        pltpu.sync_copy(src_hbm.at[pl.ds(t0, EPT2)], sidx_f.at[pl.ds(0, EPT2)])
        pltpu.sync_copy(dst_hbm.at[pl.ds(t0, EPT2)], didx_f.at[pl.ds(0, EPT2)])
        lo = cid * HALFN
        span = jnp.where(cid == 0, HALFN, N - HALFN)

        junk = JUNK0 + lax.iota(jnp.int32, 16)
        trash = CNT_AT + 16 + lax.iota(jnp.int32, 16)
        def comp(g, cnt):
            v = didx_f[pl.ds(g * 16, 16)]
            s = sidx_f[pl.ds(g * 16, 16)]
            rel = v - lo
            ok = (rel >= 0) & (rel < span)
            c = plsc.cumsum(ok.astype(jnp.int32))
            pos = jnp.where(ok, cnt + c - 1, trash)
            plsc.store_scatter(didx_f, [pos], rel)
            plsc.store_scatter(sidx_f, [pos], s)
            return cnt + c[15]
        cnt = lax.fori_loop(0, EPT2 // 16, comp, jnp.int32(0))

        for t in range(CHUNK // 16):
            didx_f[pl.ds(cnt + 16 * t, 16)] = junk
            sidx_f[pl.ds(cnt + 16 * t, 16)] = jnp.zeros((16,), jnp.int32)
        nch = (cnt + CHUNK - 1) >> 7
        for t in range(CHUNK // 16):
            didx_f[pl.ds(nch * CHUNK + 16 * t, 16)] = junk

        didx_f[pl.ds(CNT_AT, 16)] = jnp.full((16,), cnt, jnp.int32)
        pltpu.sync_copy(sidx_f, csrc_hbm.at[cid].at[sid])
        pltpu.sync_copy(didx_f, cdst_hbm.at[cid].at[sid])

        plsc.subcore_barrier()
        def hbody(j, _):
            pltpu.sync_copy(ones_v, deg_sh.at[didx_f.at[pl.ds(j * CHUNK,
                                                              CHUNK)]],
                            add=True)
            return 0
        lax.fori_loop(0, nch, hbody, 0)

        plsc.subcore_barrier()
        e0 = jnp.minimum(sid * 320, span - 320)
        pltpu.sync_copy(deg_sh.at[pl.ds(e0, 320)],
                        deg_hbm.at[pl.ds(lo + e0, 320)])

    return prep_kernel


def _make_agg(d, nb):
    @functools.partial(
        pl.kernel,
        mesh=_mesh,
        compiler_params=pltpu.CompilerParams(use_tc_tiling_on_sc=False,
                                             needs_layout_passes=False),
        out_type=jax.ShapeDtypeStruct((N, d), jnp.float32),
        scratch_types=[
            pltpu.VMEM((LSTW,), jnp.int32),
            pltpu.VMEM((LSTW,), jnp.int32),
            [pltpu.VMEM((CHUNK, d), jnp.float32)] * nb,
            pltpu.VMEM_SHARED((N_SH, d), jnp.float32),
            [pltpu.SemaphoreType.DMA] * nb,
            [pltpu.SemaphoreType.DMA] * nb,
        ],
    )
    def agg_kernel(g_hbm, csrc_hbm, cdst_hbm, out_hbm,
                   sidx_f, didx_f, rows, agg_sh, sem_g, sem_s):
        cid = lax.axis_index("c")
        sid = lax.axis_index("s")

        _fill_rows(rows[0], 64, d, 0.0)
        z0 = jnp.minimum(sid * 320, N_SH - 320)
        for b in range(5):
            pltpu.sync_copy(rows[0].at[pl.ds(0, 64)],
                            agg_sh.at[pl.ds(z0 + b * 64, 64)])

        pltpu.sync_copy(csrc_hbm.at[cid].at[sid], sidx_f)
        pltpu.sync_copy(cdst_hbm.at[cid].at[sid], didx_f)
        cnt = jnp.sum(didx_f[pl.ds(CNT_AT, 16)]) >> 4
        nch = (cnt + CHUNK - 1) >> 7
        if nb == 3:
            nfull = (nch * 43691) >> 17
        else:
            nfull = nch >> 2
        span = jnp.where(cid == 0, HALFN, N - HALFN)
        plsc.subcore_barrier()

        jchunk = didx_f.at[pl.ds(nch * CHUNK, CHUNK)]
        for b in range(nb):
            pltpu.async_copy(rows[0], agg_sh.at[jchunk], sem_s[b], add=True)

        def body(i, _):
            j = nb * i
            for b in range(nb):
                pltpu.make_async_copy(rows[b], agg_sh.at[jchunk],
                                      sem_s[b]).wait()
            gs = [pltpu.async_copy(
                      g_hbm.at[sidx_f.at[pl.ds((j + b) * CHUNK, CHUNK)]],
                      rows[b], sem_g[b])
                  for b in range(nb)]
            for b in range(nb):
                gs[b].wait()
                pltpu.async_copy(
                    rows[b],
                    agg_sh.at[didx_f.at[pl.ds((j + b) * CHUNK, CHUNK)]],
                    sem_s[b], add=True)
            return 0
        lax.fori_loop(0, nfull, body, 0)

        def tail(j, _):
            pltpu.make_async_copy(rows[0], agg_sh.at[jchunk], sem_s[0]).wait()
            pltpu.async_copy(g_hbm.at[sidx_f.at[pl.ds(j * CHUNK, CHUNK)]],
                             rows[0], sem_g[0]).wait()
            pltpu.async_copy(rows[0],
                             agg_sh.at[didx_f.at[pl.ds(j * CHUNK, CHUNK)]],
                             sem_s[0], add=True)
            return 0
        lax.fori_loop(nb * nfull, nch, tail, 0)

        for b in range(nb):
            pltpu.make_async_copy(rows[b], agg_sh.at[jchunk], sem_s[b]).wait()

        plsc.subcore_barrier()
        e0 = jnp.minimum(sid * 320, span - 320)
        lo = cid * HALFN
        pltpu.sync_copy(agg_sh.at[pl.ds(e0, 320)],
                        out_hbm.at[pl.ds(lo + e0, 320)])

    return agg_kernel


_prep_call = _make_prep()
_agg_wide = _make_agg(D_HID, 3)
_agg_narrow = _make_agg(D_OUTP, 4)

RB = 2000


def _dis_of(deg_blk):
    return lax.rsqrt(deg_blk[:, 0:1] + 1.0)


def _mm_body(x_ref, w_ref, h_ref):
    h_ref[...] = jnp.dot(x_ref[...], w_ref[...],
                         preferred_element_type=jnp.float32)


def _scale_body(h_ref, deg_ref, g_ref):
    g_ref[...] = h_ref[...] * _dis_of(deg_ref[...])


def _mid_body(agg_ref, g1_ref, deg_ref, b1_ref, w2_ref, g2_ref):
    dis = _dis_of(deg_ref[...])
    h = jnp.maximum(b1_ref[...] + dis * (agg_ref[...] + g1_ref[...]), 0.0)
    g2_ref[...] = jnp.dot(h, w2_ref[...],
                          preferred_element_type=jnp.float32) * dis


def _fin_body(agg_ref, g2_ref, deg_ref, b2_ref, o_ref):
    dis = _dis_of(deg_ref[...])
    o_ref[...] = b2_ref[...] + dis * (agg_ref[...] + g2_ref[...])


def kernel(x, edge_index, W1, b1, W2, b2):
    src = edge_index[0].astype(jnp.int32)
    dst = edge_index[1].astype(jnp.int32)
    src = jnp.concatenate([src, jnp.zeros((E_PAD - E,), jnp.int32)])
    dst = jnp.concatenate([dst, jnp.full((E_PAD - E,), PAD_DST, jnp.int32)])

    dega, csrc, cdst = _prep_call(src, dst)

    h1 = pl.pallas_call(
        _mm_body,
        grid=(N // RB,),
        in_specs=[
            pl.BlockSpec((RB, D_IN), lambda i: (i, 0)),
            pl.BlockSpec((D_IN, D_HID), lambda i: (0, 0)),
        ],
        out_specs=pl.BlockSpec((RB, D_HID), lambda i: (i, 0)),
        out_shape=jax.ShapeDtypeStruct((N, D_HID), jnp.float32),
    )(x, W1)

    deg_spec = pl.BlockSpec((RB, 16), lambda i: (i, 0))
    g1 = pl.pallas_call(
        _scale_body,
        grid=(N // RB,),
        in_specs=[
            pl.BlockSpec((RB, D_HID), lambda i: (i, 0)),
            deg_spec,
        ],
        out_specs=pl.BlockSpec((RB, D_HID), lambda i: (i, 0)),
        out_shape=jax.ShapeDtypeStruct((N, D_HID), jnp.float32),
    )(h1, dega)

    agg1 = _agg_wide(g1, csrc, cdst)

    b1r = b1.reshape(1, D_HID)
    w2p = jnp.pad(W2, ((0, 0), (0, D_OUTP - D_OUT)))
    g2 = pl.pallas_call(
        _mid_body,
        grid=(N // RB,),
        in_specs=[
            pl.BlockSpec((RB, D_HID), lambda i: (i, 0)),
            pl.BlockSpec((RB, D_HID), lambda i: (i, 0)),
            deg_spec,
            pl.BlockSpec((1, D_HID), lambda i: (0, 0)),
            pl.BlockSpec((D_HID, D_OUTP), lambda i: (0, 0)),
        ],
        out_specs=pl.BlockSpec((RB, D_OUTP), lambda i: (i, 0)),
        out_shape=jax.ShapeDtypeStruct((N, D_OUTP), jnp.float32),
    )(agg1, g1, dega, b1r, w2p)

    agg2 = _agg_narrow(g2, csrc, cdst)

    b2p = jnp.pad(b2, (0, D_OUTP - D_OUT)).reshape(1, D_OUTP)
    out = pl.pallas_call(
        _fin_body,
        grid=(N // RB,),
        in_specs=[
            pl.BlockSpec((RB, D_OUTP), lambda i: (i, 0)),
            pl.BlockSpec((RB, D_OUTP), lambda i: (i, 0)),
            deg_spec,
            pl.BlockSpec((1, D_OUTP), lambda i: (0, 0)),
        ],
        out_specs=pl.BlockSpec((RB, D_OUTP), lambda i: (i, 0)),
        out_shape=jax.ShapeDtypeStruct((N, D_OUTP), jnp.float32),
    )(agg2, g2, dega, b2p)

    return out[:, :D_OUT]


# speedup vs baseline: 1.0833x; 1.0147x over previous
import functools

import jax
import jax.numpy as jnp
from jax import lax
from jax.experimental import pallas as pl
from jax.experimental.pallas import tpu as pltpu
from jax.experimental.pallas import tpu_sc as plsc

N = 10000
D_IN = 128
D_HID = 128
D_OUT = 40
D_OUTP = 48

NC, NS = 2, 16
NW = NC * NS
CHUNK = 128
E = 320000
CPW = -(-E // (NW * CHUNK))
CPW += CPW % 2
E_PAD = NW * CPW * CHUNK
NCHUNKS = E_PAD // CHUNK
PAD_DST = N

HALFN = 5008
N_SH = 5024
JUNK0 = 5008
CPW2 = NCHUNKS // NS
EPT2 = CPW2 * CHUNK
LSTW = EPT2 + 2 * CHUNK
CNT_AT = EPT2 + CHUNK

_mesh = plsc.VectorSubcoreMesh(core_axis_name="c", subcore_axis_name="s")


def _fill_rows(buf, rows, d, value):
    def body(r, _):
        for k in range(d // 16):
            buf[r, pl.ds(k * 16, 16)] = jnp.full((16,), value, jnp.float32)
        return 0
    lax.fori_loop(0, rows, body, 0)


def _make_prep():
    @functools.partial(
        pl.kernel,
        mesh=_mesh,
        compiler_params=pltpu.CompilerParams(use_tc_tiling_on_sc=False,
                                             needs_layout_passes=False),
        out_type=(
            jax.ShapeDtypeStruct((N, 16), jnp.float32),
            jax.ShapeDtypeStruct((NC, NS, LSTW), jnp.int32),
            jax.ShapeDtypeStruct((NC, NS, LSTW), jnp.int32),
        ),
        scratch_types=[
            pltpu.VMEM((LSTW,), jnp.int32),
            pltpu.VMEM((LSTW,), jnp.int32),
            pltpu.VMEM((CHUNK, 16), jnp.float32),
            pltpu.VMEM((CHUNK, 16), jnp.float32),
            pltpu.VMEM_SHARED((N_SH, 16), jnp.float32),
        ],
    )
    def prep_kernel(src_hbm, dst_hbm, deg_hbm, csrc_hbm, cdst_hbm,
                    sidx_f, didx_f, ones_v, zeros_v, deg_sh):
        cid = lax.axis_index("c")
        sid = lax.axis_index("s")

        _fill_rows(ones_v, CHUNK, 16, 1.0)
        _fill_rows(zeros_v, CHUNK, 16, 0.0)

        z0 = jnp.minimum(sid * 320, N_SH - 320)
        for b in range(5):
            pltpu.sync_copy(zeros_v.at[pl.ds(0, 64)],
                            deg_sh.at[pl.ds(z0 + b * 64, 64)])

        t0 = sid * EPT2
        pltpu.sync_copy(src_hbm.at[pl.ds(t0, EPT2)], sidx_f.at[pl.ds(0, EPT2)])
        pltpu.sync_copy(dst_hbm.at[pl.ds(t0, EPT2)], didx_f.at[pl.ds(0, EPT2)])
        lo = cid * HALFN
        span = jnp.where(cid == 0, HALFN, N - HALFN)

        junk = JUNK0 + lax.iota(jnp.int32, 16)
        trash = CNT_AT + 16 + lax.iota(jnp.int32, 16)
        def comp(g, cnt):
            v = didx_f[pl.ds(g * 16, 16)]
            s = sidx_f[pl.ds(g * 16, 16)]
            rel = v - lo
            ok = (rel >= 0) & (rel < span)
            c = plsc.cumsum(ok.astype(jnp.int32))
            pos = jnp.where(ok, cnt + c - 1, trash)
            plsc.store_scatter(didx_f, [pos], rel)
            plsc.store_scatter(sidx_f, [pos], s)
            return cnt + c[15]
        cnt = lax.fori_loop(0, EPT2 // 16, comp, jnp.int32(0))

        for t in range(CHUNK // 16):
            didx_f[pl.ds(cnt + 16 * t, 16)] = junk
            sidx_f[pl.ds(cnt + 16 * t, 16)] = jnp.zeros((16,), jnp.int32)
        nch = (cnt + CHUNK - 1) >> 7
        for t in range(CHUNK // 16):
            didx_f[pl.ds(nch * CHUNK + 16 * t, 16)] = junk

        didx_f[pl.ds(CNT_AT, 16)] = jnp.full((16,), cnt, jnp.int32)
        pltpu.sync_copy(sidx_f, csrc_hbm.at[cid].at[sid])
        pltpu.sync_copy(didx_f, cdst_hbm.at[cid].at[sid])

        plsc.subcore_barrier()
        def hbody(j, _):
            pltpu.sync_copy(ones_v, deg_sh.at[didx_f.at[pl.ds(j * CHUNK,
                                                              CHUNK)]],
                            add=True)
            return 0
        lax.fori_loop(0, nch, hbody, 0)

        plsc.subcore_barrier()
        e0 = jnp.minimum(sid * 320, span - 320)
        pltpu.sync_copy(deg_sh.at[pl.ds(e0, 320)],
                        deg_hbm.at[pl.ds(lo + e0, 320)])

    return prep_kernel


def _make_agg(d, nb):
    @functools.partial(
        pl.kernel,
        mesh=_mesh,
        compiler_params=pltpu.CompilerParams(use_tc_tiling_on_sc=False,
                                             needs_layout_passes=False),
        out_type=jax.ShapeDtypeStruct((N, d), jnp.float32),
        scratch_types=[
            pltpu.VMEM((LSTW,), jnp.int32),
            pltpu.VMEM((LSTW,), jnp.int32),
            [pltpu.VMEM((CHUNK, d), jnp.float32)] * nb,
            pltpu.VMEM_SHARED((N_SH, d), jnp.float32),
            [pltpu.SemaphoreType.DMA] * nb,
            [pltpu.SemaphoreType.DMA] * nb,
        ],
    )
    def agg_kernel(g_hbm, csrc_hbm, cdst_hbm, out_hbm,
                   sidx_f, didx_f, rows, agg_sh, sem_g, sem_s):
        cid = lax.axis_index("c")
        sid = lax.axis_index("s")

        _fill_rows(rows[0], 64, d, 0.0)
        z0 = jnp.minimum(sid * 320, N_SH - 320)
        for b in range(5):
            pltpu.sync_copy(rows[0].at[pl.ds(0, 64)],
                            agg_sh.at[pl.ds(z0 + b * 64, 64)])

        pltpu.sync_copy(csrc_hbm.at[cid].at[sid], sidx_f)
        pltpu.sync_copy(cdst_hbm.at[cid].at[sid], didx_f)
        cnt = jnp.sum(didx_f[pl.ds(CNT_AT, 16)]) >> 4
        nch = (cnt + CHUNK - 1) >> 7
        if nb == 3:
            nfull = (nch * 43691) >> 17
        else:
            nfull = nch >> 2
        span = jnp.where(cid == 0, HALFN, N - HALFN)
        plsc.subcore_barrier()

        jchunk = didx_f.at[pl.ds(nch * CHUNK, CHUNK)]
        for b in range(nb):
            pltpu.async_copy(rows[0], agg_sh.at[jchunk], sem_s[b], add=True)

        def body(i, _):
            j = nb * i
            for b in range(nb):
                pltpu.make_async_copy(rows[b], agg_sh.at[jchunk],
                                      sem_s[b]).wait()
            gs = [pltpu.async_copy(
                      g_hbm.at[sidx_f.at[pl.ds((j + b) * CHUNK, CHUNK)]],
                      rows[b], sem_g[b])
                  for b in range(nb)]
            for b in range(nb):
                gs[b].wait()
                pltpu.async_copy(
                    rows[b],
                    agg_sh.at[didx_f.at[pl.ds((j + b) * CHUNK, CHUNK)]],
                    sem_s[b], add=True)
            return 0
        lax.fori_loop(0, nfull, body, 0)

        def tail(j, _):
            pltpu.make_async_copy(rows[0], agg_sh.at[jchunk], sem_s[0]).wait()
            pltpu.async_copy(g_hbm.at[sidx_f.at[pl.ds(j * CHUNK, CHUNK)]],
                             rows[0], sem_g[0]).wait()
            pltpu.async_copy(rows[0],
                             agg_sh.at[didx_f.at[pl.ds(j * CHUNK, CHUNK)]],
                             sem_s[0], add=True)
            return 0
        lax.fori_loop(nb * nfull, nch, tail, 0)

        for b in range(nb):
            pltpu.make_async_copy(rows[b], agg_sh.at[jchunk], sem_s[b]).wait()

        plsc.subcore_barrier()
        e0 = jnp.minimum(sid * 320, span - 320)
        lo = cid * HALFN
        pltpu.sync_copy(agg_sh.at[pl.ds(e0, 320)],
                        out_hbm.at[pl.ds(lo + e0, 320)])

    return agg_kernel


_prep_call = _make_prep()
_agg_wide = _make_agg(D_HID, 3)
_agg_narrow = _make_agg(D_OUTP, 4)

RB = 5000


def _dis_of(deg_blk):
    return lax.rsqrt(deg_blk[:, 0:1] + 1.0)


def _mm_body(x_ref, w_ref, h_ref):
    h_ref[...] = jnp.dot(x_ref[...], w_ref[...],
                         preferred_element_type=jnp.float32)


def _scale_body(h_ref, deg_ref, g_ref):
    g_ref[...] = h_ref[...] * _dis_of(deg_ref[...])


def _mid_body(agg_ref, g1_ref, deg_ref, b1_ref, w2_ref, g2_ref):
    dis = _dis_of(deg_ref[...])
    h = jnp.maximum(b1_ref[...] + dis * (agg_ref[...] + g1_ref[...]), 0.0)
    g2_ref[...] = jnp.dot(h, w2_ref[...],
                          preferred_element_type=jnp.float32) * dis


def _fin_body(agg_ref, g2_ref, deg_ref, b2_ref, o_ref):
    dis = _dis_of(deg_ref[...])
    o_ref[...] = b2_ref[...] + dis * (agg_ref[...] + g2_ref[...])


def kernel(x, edge_index, W1, b1, W2, b2):
    src = edge_index[0].astype(jnp.int32)
    dst = edge_index[1].astype(jnp.int32)
    src = jnp.concatenate([src, jnp.zeros((E_PAD - E,), jnp.int32)])
    dst = jnp.concatenate([dst, jnp.full((E_PAD - E,), PAD_DST, jnp.int32)])

    dega, csrc, cdst = _prep_call(src, dst)

    h1 = pl.pallas_call(
        _mm_body,
        grid=(N // RB,),
        in_specs=[
            pl.BlockSpec((RB, D_IN), lambda i: (i, 0)),
            pl.BlockSpec((D_IN, D_HID), lambda i: (0, 0)),
        ],
        out_specs=pl.BlockSpec((RB, D_HID), lambda i: (i, 0)),
        out_shape=jax.ShapeDtypeStruct((N, D_HID), jnp.float32),
    )(x, W1)

    deg_spec = pl.BlockSpec((RB, 16), lambda i: (i, 0))
    g1 = pl.pallas_call(
        _scale_body,
        grid=(N // RB,),
        in_specs=[
            pl.BlockSpec((RB, D_HID), lambda i: (i, 0)),
            deg_spec,
        ],
        out_specs=pl.BlockSpec((RB, D_HID), lambda i: (i, 0)),
        out_shape=jax.ShapeDtypeStruct((N, D_HID), jnp.float32),
    )(h1, dega)

    agg1 = _agg_wide(g1, csrc, cdst)

    b1r = b1.reshape(1, D_HID)
    w2p = jnp.pad(W2, ((0, 0), (0, D_OUTP - D_OUT)))
    g2 = pl.pallas_call(
        _mid_body,
        grid=(N // RB,),
        in_specs=[
            pl.BlockSpec((RB, D_HID), lambda i: (i, 0)),
            pl.BlockSpec((RB, D_HID), lambda i: (i, 0)),
            deg_spec,
            pl.BlockSpec((1, D_HID), lambda i: (0, 0)),
            pl.BlockSpec((D_HID, D_OUTP), lambda i: (0, 0)),
        ],
        out_specs=pl.BlockSpec((RB, D_OUTP), lambda i: (i, 0)),
        out_shape=jax.ShapeDtypeStruct((N, D_OUTP), jnp.float32),
    )(agg1, g1, dega, b1r, w2p)

    agg2 = _agg_narrow(g2, csrc, cdst)

    b2p = jnp.pad(b2, (0, D_OUTP - D_OUT)).reshape(1, D_OUTP)
    out = pl.pallas_call(
        _fin_body,
        grid=(N // RB,),
        in_specs=[
            pl.BlockSpec((RB, D_OUTP), lambda i: (i, 0)),
            pl.BlockSpec((RB, D_OUTP), lambda i: (i, 0)),
            deg_spec,
            pl.BlockSpec((1, D_OUTP), lambda i: (0, 0)),
        ],
        out_specs=pl.BlockSpec((RB, D_OUTP), lambda i: (i, 0)),
        out_shape=jax.ShapeDtypeStruct((N, D_OUTP), jnp.float32),
    )(agg2, g2, dega, b2p)

    return out[:, :D_OUT]
